# Initial kernel scaffold; baseline (speedup 1.0000x reference)
import jax
import jax.numpy as jnp
from jax.experimental import pallas as pl


def kernel(x, edge_index, batch, W1, b1, Wk, bk, Wq, bq, Wv, bv, Ws, bs, Wih, Whh, bih, bhh, gamma, beta, Wg, att_src, att_dst, bg, Wih_m, Whh_m, bih_m, bhh_m, W2, b2):
    raise NotImplementedError("write your pallas kernel here")


# speedup vs baseline: 1.2156x; 1.2156x over previous
import jax
import jax.numpy as jnp
from jax import lax
from jax.experimental import pallas as pl
from jax.experimental.pallas import tpu as pltpu
from jax.experimental.pallas import tpu_sc as plsc

H = 97
HP = 112
QVW = 2 * HP
SUB = 128
CH = 2048
STRIPE = 552
RANGE = STRIPE * 16
NRANGE = 6
NPAD = RANGE * NRANGE
NB = 1000
F32 = jnp.float32
I32 = jnp.int32


def _edge_body_factory(nchunk, epw):
    def body(srcp_hbm, dstp_hbm, k_hbm, qv_hbm, agg_hbm,
             dstbuf, srcbuf, csrc, cdstg, cdst3, kbuf, qvbuf, msgbuf,
             aggsp, sem1, sem2):
        cid = lax.axis_index("c")
        sid = lax.axis_index("s")
        ebase = sid * epw
        row0 = sid * STRIPE
        zeros16 = jnp.zeros((16,), I32)

        for rr in range(NRANGE // 2):
            r = cid * (NRANGE // 2) + rr
            lo = r * RANGE

            def zrow(i, carry):
                for v7 in range(HP // 16):
                    msgbuf[0, i, pl.ds(v7 * 16, 16)] = jnp.zeros((16,), F32)
                return carry
            lax.fori_loop(0, SUB, zrow, 0)
            for b in range(STRIPE // SUB):
                pltpu.sync_copy(msgbuf,
                                aggsp.at[:, pl.ds(row0 + b * SUB, SUB)])
            tail = STRIPE % SUB
            if tail:
                pltpu.sync_copy(
                    msgbuf.at[:, pl.ds(0, tail)],
                    aggsp.at[:, pl.ds(row0 + (STRIPE // SUB) * SUB, tail)])
            plsc.subcore_barrier()

            def chunk_body(c, carry):
                base = ebase + c * CH
                pltpu.sync_copy(dstp_hbm.at[pl.ds(base, CH)], dstbuf)
                pltpu.sync_copy(srcp_hbm.at[pl.ds(base, CH)], srcbuf)

                def cvec(i, n):
                    d = dstbuf[pl.ds(i * 16, 16)]
                    s = srcbuf[pl.ds(i * 16, 16)]
                    m = (d >= lo) & (d < lo + RANGE)
                    mi = m.astype(I32)
                    pos = n + plsc.cumsum(mi) - 1
                    plsc.store_scatter(csrc, [pos], s, mask=m)
                    plsc.store_scatter(cdstg, [pos], d, mask=m)
                    plsc.store_scatter(
                        cdst3, [pos >> 7, zeros16, pos & 127], d - lo, mask=m)
                    return n + jnp.sum(mi)
                n = lax.fori_loop(0, CH // 16, cvec, 0)

                nb0 = (n // SUB) * SUB
                for j in range(SUB // 16):
                    p = nb0 + j * 16 + lax.iota(I32, 16)
                    pm = p >= n
                    plsc.store_scatter(csrc, [p], zeros16, mask=pm)
                    plsc.store_scatter(cdstg, [p], zeros16, mask=pm)
                    plsc.store_scatter(
                        cdst3, [p >> 7, zeros16, p & 127],
                        jnp.full((16,), RANGE, I32), mask=pm)
                nsub = (n + SUB - 1) // SUB

                def sub_body(j, carry):
                    cp1 = pltpu.async_copy(
                        k_hbm.at[cdstg.at[pl.ds(j * SUB, SUB)]], kbuf, sem1)
                    cp2 = pltpu.async_copy(
                        qv_hbm.at[csrc.at[pl.ds(j * SUB, SUB)]], qvbuf, sem2)
                    cp1.wait()
                    cp2.wait()

                    def row(i, c2):
                        for v7 in range(HP // 16):
                            kv = kbuf[i, pl.ds(v7 * 16, 16)]
                            qq = qvbuf[i, pl.ds(v7 * 16, 16)]
                            vv = qvbuf[i, pl.ds(HP + v7 * 16, 16)]
                            g = 1.0 / (1.0 + jnp.exp(-(kv + qq)))
                            msgbuf[0, i, pl.ds(v7 * 16, 16)] = g * vv
                        return c2
                    lax.fori_loop(0, SUB, row, 0)
                    pltpu.sync_copy(msgbuf, aggsp.at[cdst3.at[j]], add=True)
                    return carry
                lax.fori_loop(0, nsub, sub_body, 0)
                return carry
            lax.fori_loop(0, nchunk, chunk_body, 0)

            plsc.subcore_barrier()
            hbase = lo + row0
            for b in range(STRIPE // SUB):
                pltpu.sync_copy(aggsp.at[:, pl.ds(row0 + b * SUB, SUB)],
                                agg_hbm.at[:, pl.ds(hbase + b * SUB, SUB)])
            if tail:
                pltpu.sync_copy(
                    aggsp.at[:, pl.ds(row0 + (STRIPE // SUB) * SUB, tail)],
                    agg_hbm.at[:, pl.ds(hbase + (STRIPE // SUB) * SUB, tail)])
            plsc.subcore_barrier()
    return body


def _make_edge_call(ep):
    epw = ep // 16
    nchunk = epw // CH
    mesh = plsc.VectorSubcoreMesh(
        core_axis_name="c", subcore_axis_name="s", num_cores=2,
        num_subcores=16)
    return pl.kernel(
        _edge_body_factory(nchunk, epw),
        out_type=jax.ShapeDtypeStruct((1, NPAD, HP), F32),
        mesh=mesh,
        compiler_params=pltpu.CompilerParams(
            needs_layout_passes=False, use_tc_tiling_on_sc=False),
        scratch_types=[
            pltpu.VMEM((CH,), I32),
            pltpu.VMEM((CH,), I32),
            pltpu.VMEM((CH + SUB,), I32),
            pltpu.VMEM((CH + SUB,), I32),
            pltpu.VMEM((CH // SUB + 1, 1, SUB), I32),
            pltpu.VMEM((SUB, HP), F32),
            pltpu.VMEM((SUB, QVW), F32),
            pltpu.VMEM((1, SUB, HP), F32),
            pltpu.VMEM_SHARED((1, RANGE + 16, HP), F32),
            pltpu.SemaphoreType.DMA,
            pltpu.SemaphoreType.DMA,
        ],
    )


def _k0_body(x_ref, w_ref, b_ref, o_ref):
    t = jnp.dot(x_ref[...], w_ref[...], preferred_element_type=F32) + b_ref[...]
    o_ref[...] = jnp.where(t > 0, t, 0.01 * t)


def _ka_body(xh_ref, wk_ref, wqv_ref, bk_ref, bqv_ref, k_ref, qv_ref):
    xb = xh_ref[...]
    k_ref[...] = jnp.dot(xb, wk_ref[...], preferred_element_type=F32) + bk_ref[...]
    qv_ref[...] = jnp.dot(xb, wqv_ref[...], preferred_element_type=F32) + bqv_ref[...]


def _kb_body(agg_ref, xh_ref, ws_ref, bs_ref, gsc_ref, gbe_ref,
             wih_ref, whh_ref, bih_ref, bhh_ref, o_ref):
    xb = xh_ref[...]
    s = jnp.dot(xb, ws_ref[...], preferred_element_type=F32) + bs_ref[...]
    t = agg_ref[0] + s
    h = jnp.where(t > 0, t, jnp.exp(jnp.minimum(t, 0.0)) - 1.0)
    h = h * gsc_ref[...] + gbe_ref[...]
    gi = jnp.dot(h, wih_ref[...], preferred_element_type=F32) + bih_ref[...]
    gh = jnp.dot(xb, whh_ref[...], preferred_element_type=F32) + bhh_ref[...]
    r = 1.0 / (1.0 + jnp.exp(-(gi[:, 0:128] + gh[:, 0:128])))
    z = 1.0 / (1.0 + jnp.exp(-(gi[:, 128:256] + gh[:, 128:256])))
    nn = jnp.tanh(gi[:, 256:384] + r * gh[:, 256:384])
    res = (1.0 - z[:, 0:97]) * nn[:, 0:97] + z[:, 0:97] * xb
    o_ref[...] = jnp.maximum(res, 0.0)


def _final_linear_kernel(out_ref, w_ref, b_ref, o_ref):
    t = out_ref[...] * w_ref[...]
    o_ref[...] = jnp.sum(t, axis=1, keepdims=True) + b_ref[...]


def _padw(w, rows, cols):
    out = jnp.zeros((rows, cols), F32)
    return out.at[: w.shape[0], : w.shape[1]].set(w)


def kernel(x, edge_index, batch, W1, b1, Wk, bk, Wq, bq, Wv, bv, Ws, bs,
           Wih, Whh, bih, bhh, gamma, beta, Wg, att_src, att_dst, bg,
           Wih_m, Whh_m, bih_m, bhh_m, W2, b2):
    N = x.shape[0]
    E = edge_index.shape[1]
    B = 2048
    L = Wk.shape[0]
    grid = (N + NB - 1) // NB

    ep = ((E + 16 * CH - 1) // (16 * CH)) * (16 * CH)
    src = edge_index[0]
    dst = edge_index[1]
    srcp = jnp.concatenate([src, jnp.zeros((ep - E,), I32)])
    dstp = jnp.concatenate([dst, jnp.full((ep - E,), NPAD, I32)])
    edge_call = _make_edge_call(ep)

    w1t = W1.T
    b1r = b1[None]
    kas, kbs = [], []
    for li in range(L):
        wkt = _padw(Wk[li].T, H, HP)
        wqvt = jnp.concatenate(
            [_padw(Wq[li].T, H, HP), _padw(Wv[li].T, H, HP)], axis=1)
        bkp = _padw(bk[li][None], 1, HP)
        bqvp = jnp.concatenate(
            [_padw(bq[li][None], 1, HP), _padw(bv[li][None], 1, HP)], axis=1)
        kas.append((wkt, wqvt, bkp, bqvp))

        wst = _padw(Ws[li].T, H, HP)
        bsp = _padw(bs[li][None], 1, HP)
        if li == 0:
            gsc = _padw(jnp.ones((1, H), F32), 1, HP)
            gbe = jnp.zeros((1, HP), F32)
        else:
            gsc = _padw((gamma[li - 1] / jnp.sqrt(1.0 + 1e-05))[None], 1, HP)
            gbe = _padw(beta[li - 1][None], 1, HP)
        wiht = jnp.zeros((HP, 384), F32)
        whht = jnp.zeros((H, 384), F32)
        bihp = jnp.zeros((1, 384), F32)
        bhhp = jnp.zeros((1, 384), F32)
        for g in range(3):
            wiht = wiht.at[:H, g * 128: g * 128 + H].set(Wih[li][g * H:(g + 1) * H].T)
            whht = whht.at[:, g * 128: g * 128 + H].set(Whh[li][g * H:(g + 1) * H].T)
            bihp = bihp.at[0, g * 128: g * 128 + H].set(bih[li][g * H:(g + 1) * H])
            bhhp = bhhp.at[0, g * 128: g * 128 + H].set(bhh[li][g * H:(g + 1) * H])
        kbs.append((wst, bsp, gsc, gbe, wiht, whht, bihp, bhhp))

    full = lambda shp: pl.BlockSpec(shp, lambda i: (0, 0))
    rows = lambda w: pl.BlockSpec((NB, w), lambda i: (i, 0))

    xh = pl.pallas_call(
        _k0_body, grid=(grid,),
        in_specs=[rows(128), full((128, H)), full((1, H))],
        out_specs=rows(H),
        out_shape=jax.ShapeDtypeStruct((N, H), F32),
    )(x, w1t, b1r)

    ka_call = pl.pallas_call(
        _ka_body, grid=(grid,),
        in_specs=[rows(H), full((H, HP)), full((H, QVW)), full((1, HP)),
                  full((1, QVW))],
        out_specs=[rows(HP), rows(QVW)],
        out_shape=[jax.ShapeDtypeStruct((N, HP), F32),
                   jax.ShapeDtypeStruct((N, QVW), F32)],
    )
    kb_call = pl.pallas_call(
        _kb_body, grid=(grid,),
        in_specs=[pl.BlockSpec((1, NB, HP), lambda i: (0, i, 0)),
                  rows(H), full((H, HP)), full((1, HP)),
                  full((1, HP)), full((1, HP)), full((HP, 384)),
                  full((H, 384)), full((1, 384)), full((1, 384))],
        out_specs=rows(H),
        out_shape=jax.ShapeDtypeStruct((N, H), F32),
    )

    for li in range(L):
        kk, qv = ka_call(xh, *kas[li])
        agg = edge_call(srcp, dstp, kk, qv)
        xh = kb_call(agg, xh, *kbs[li])

    out = jax.nn.relu(jax.ops.segment_sum(xh, batch, num_segments=B))
    xs = xh @ Wg.T
    a_src = jnp.sum(xs * att_src, axis=-1)
    for _ in range(7):
        xd = out @ Wg.T
        a_dst = jnp.sum(xd * att_dst, axis=-1)
        alpha = jax.nn.leaky_relu(a_src + a_dst[batch], 0.01)
        m = jax.ops.segment_max(alpha, batch, num_segments=B)
        m = jnp.where(jnp.isfinite(m), m, 0.0)
        e = jnp.exp(alpha - m[batch])
        denom = jax.ops.segment_sum(e, batch, num_segments=B)
        att = e / (denom[batch] + 1e-16)
        hmol = jax.ops.segment_sum(xs * att[:, None], batch, num_segments=B) + bg
        hmol = jax.nn.elu(hmol)
        gi = hmol @ Wih_m.T + bih_m
        gh = out @ Whh_m.T + bhh_m
        ir, iz, inn = jnp.split(gi, 3, axis=1)
        hr, hz, hn = jnp.split(gh, 3, axis=1)
        rg = jax.nn.sigmoid(ir + hr)
        zg = jax.nn.sigmoid(iz + hz)
        ng = jnp.tanh(inn + rg * hn)
        out = jax.nn.relu((1.0 - zg) * ng + zg * out)

    return pl.pallas_call(
        _final_linear_kernel,
        out_shape=jax.ShapeDtypeStruct((B, 1), F32),
    )(out, W2, b2[None])
